# baseline (speedup 1.0000x reference)
import functools

import jax
import jax.numpy as jnp
from jax import lax
from jax.experimental import pallas as pl
from jax.experimental.pallas import tpu as pltpu
from jax.experimental.pallas import tpu_sc as plsc

NN = 10000
EE = 320000
NC = 2
NS = 16
NW = NC * NS
EB = 125
ER = EE // EB
RPT = ER // NW
CH = 632
CHL = NN - 15 * CH

_MESH = plsc.VectorSubcoreMesh(core_axis_name="c", subcore_axis_name="s")


_DCH = 640


def _deg_body(dst_hbm, out_hbm, dst_v, ones_v, chunk_v, deg_sh):
    cid = lax.axis_index("c")
    sid = lax.axis_index("s")
    wid = sid * NC + cid

    for i in range(_DCH // 16):
        chunk_v[pl.ds(i * 16, 16)] = jnp.zeros((16,), jnp.float32)
    pltpu.sync_copy(chunk_v, deg_sh.at[pl.ds(sid * _DCH, _DCH)])
    plsc.subcore_barrier()

    pltpu.sync_copy(dst_hbm.at[pl.ds(wid * RPT, RPT)], dst_v)
    for i in range(8):
        ones_v[pl.ds(i * 16, 16)] = jnp.ones((16,), jnp.float32)

    def body(j, carry):
        pltpu.sync_copy(ones_v.at[pl.ds(0, EB)], deg_sh.at[dst_v.at[j]],
                        add=True)
        return carry

    lax.fori_loop(0, RPT, body, 0)
    plsc.subcore_barrier()

    pltpu.sync_copy(deg_sh.at[pl.ds(sid * _DCH, _DCH)], chunk_v)

    @pl.when(sid < 15)
    def _():
        pltpu.sync_copy(chunk_v, out_hbm.at[pl.ds(cid * NN + sid * _DCH,
                                                  _DCH)])

    @pl.when(sid == 15)
    def _():
        pltpu.sync_copy(chunk_v.at[pl.ds(0, NN - 15 * _DCH)],
                        out_hbm.at[pl.ds(cid * NN + 15 * _DCH,
                                         NN - 15 * _DCH)])


_deg_kernel = pl.kernel(
    _deg_body,
    out_type=jax.ShapeDtypeStruct((NC * NN,), jnp.float32),
    mesh=_MESH,
    scratch_types=[
        pltpu.VMEM((RPT, EB), jnp.int32),
        pltpu.VMEM((128,), jnp.float32),
        pltpu.VMEM((_DCH,), jnp.float32),
        pltpu.VMEM_SHARED((NS * _DCH,), jnp.float32),
    ],
)


_NBT = 8


def _make_scat(width):
    def body(hs_hbm, src_hbm, dst_hbm, zeros_hbm, out_hbm,
             src_v, dst_v, rows_v, gsem, acc_sh):
        cid = lax.axis_index("c")
        sid = lax.axis_index("s")
        wid = sid * NC + cid

        @pl.when(sid < 15)
        def _():
            pltpu.sync_copy(zeros_hbm.at[pl.ds(sid * CH, CH)],
                            acc_sh.at[pl.ds(sid * CH, CH)])

        @pl.when(sid == 15)
        def _():
            pltpu.sync_copy(zeros_hbm.at[pl.ds(15 * CH, CHL)],
                            acc_sh.at[pl.ds(15 * CH, CHL)])

        pltpu.sync_copy(src_hbm.at[pl.ds(wid * RPT, RPT)], src_v)
        pltpu.sync_copy(dst_hbm.at[pl.ds(wid * RPT, RPT)], dst_v)
        plsc.subcore_barrier()

        def gstart(j, b):
            pltpu.async_copy(hs_hbm.at[src_v.at[j]], rows_v.at[b],
                             gsem.at[b])

        def gwait(j, b):
            pltpu.make_async_copy(hs_hbm.at[src_v.at[j]], rows_v.at[b],
                                  gsem.at[b]).wait()

        for b in range(_NBT):
            gstart(b, b)

        def rnd(r, carry):
            base = r * _NBT
            for b in range(_NBT):
                j = base + b
                gwait(j, b)
                pltpu.sync_copy(rows_v.at[b], acc_sh.at[dst_v.at[j]],
                                add=True)
                jn = j + _NBT

                @pl.when(jn < RPT)
                def _():
                    gstart(jn, b)
            return carry

        lax.fori_loop(0, RPT // _NBT, rnd, 0)
        plsc.subcore_barrier()

        @pl.when(sid < 15)
        def _():
            pltpu.sync_copy(acc_sh.at[pl.ds(sid * CH, CH)],
                            out_hbm.at[pl.ds(cid * NN + sid * CH, CH)])

        @pl.when(sid == 15)
        def _():
            pltpu.sync_copy(acc_sh.at[pl.ds(15 * CH, CHL)],
                            out_hbm.at[pl.ds(cid * NN + 15 * CH, CHL)])

    return pl.kernel(
        body,
        out_type=jax.ShapeDtypeStruct((NC * NN, width), jnp.float32),
        mesh=_MESH,
        compiler_params=pltpu.CompilerParams(use_tc_tiling_on_sc=False),
        scratch_types=[
            pltpu.VMEM((RPT, EB), jnp.int32),
            pltpu.VMEM((RPT, EB), jnp.int32),
            pltpu.VMEM((_NBT, EB, width), jnp.float32),
            pltpu.SemaphoreType.DMA((_NBT,)),
            pltpu.VMEM_SHARED((NN, width), jnp.float32),
        ],
    )


_scat64 = _make_scat(64)
_scat32 = _make_scat(32)


def _tc_a1(x_ref, w1_ref, ws1_ref, bs1_ref, h1m_ref, xs_ref):
    h1m_ref[...] = jnp.dot(x_ref[...], w1_ref[...])
    xs_ref[...] = jnp.dot(x_ref[...], ws1_ref[...]) + bs1_ref[...]


def _tc_a2(h1m_ref, degp_ref, dinv_ref, hs1_ref):
    deg = degp_ref[:, 0:1] + degp_ref[:, 1:2] + 1.0
    dinv = lax.rsqrt(deg)
    dinv_ref[...] = dinv
    hs1_ref[...] = h1m_ref[...] * dinv


def _bn(h, g, be):
    m = jnp.mean(h, axis=0, keepdims=True)
    v = jnp.mean((h - m) ** 2, axis=0, keepdims=True)
    return (h - m) * lax.rsqrt(v + 1e-5) * g + be


def _accsum(ref):
    return ref[pl.ds(0, NN), :] + ref[pl.ds(NN, NN), :]


def _tc_b(acc_ref, hs1_ref, xs_ref, dinv_ref, b1_ref, g1_ref, be1_ref,
          w2_ref, hs2_ref, h1_ref):
    dinv = dinv_ref[...]
    conv = dinv * (_accsum(acc_ref) + hs1_ref[...]) + b1_ref[...]
    h = _bn(jnp.maximum(conv, 0.0), g1_ref[...], be1_ref[...])
    h1 = h + xs_ref[...]
    h1_ref[...] = h1
    hs2_ref[...] = jnp.dot(h1, w2_ref[...]) * dinv


def _tc_h1s(h1_ref, ws2_ref, bs2_ref, h1s_ref):
    h1s_ref[...] = jnp.dot(h1_ref[...], ws2_ref[...]) + bs2_ref[...]


def _tc_c(acc_ref, hs2_ref, dinv_ref, b2_ref, g2_ref, be2_ref, w3_ref,
          hs3_ref):
    dinv = dinv_ref[...]
    conv = dinv * (_accsum(acc_ref) + hs2_ref[...]) + b2_ref[...]
    h2 = _bn(jnp.maximum(conv, 0.0), g2_ref[...], be2_ref[...])
    hs3_ref[...] = jnp.dot(h2, w3_ref[...]) * dinv


_BM = 1024
_BN = 1024
_NGI = pl.cdiv(NN, _BM)
_NGJ = pl.cdiv(NN, _BN)


_ZPAD = max(_NGI * _BM, _NGJ * _BN) - NN


def _tc_decode(acc_ref, hs3_ref, dinv_ref, b3_ref, h1s_ref,
               adj_ref, zout_ref, z_v):
    i = pl.program_id(0)
    j = pl.program_id(1)

    @pl.when((i == 0) & (j == 0))
    def _():
        z = (dinv_ref[...] * (_accsum(acc_ref) + hs3_ref[...])
             + b3_ref[...] + h1s_ref[...])
        z_v[pl.ds(0, NN), :] = z
        z_v[pl.ds(NN, _ZPAD), :] = jnp.zeros((_ZPAD, 32), jnp.float32)
        zout_ref[...] = z

    zi = z_v[pl.ds(i * _BM, _BM), :]
    zj = z_v[pl.ds(j * _BN, _BN), :]
    prod = lax.dot_general(zi, zj, (((1,), (1,)), ((), ())))
    adj_ref[...] = jax.nn.sigmoid(prod)


def _f32(shape):
    return jax.ShapeDtypeStruct(shape, jnp.float32)


def kernel(x, edge_index, W1, b1, W2, b2, W3, b3, Ws1, bs1, Ws2, bs2,
           g1, be1, g2, be2):
    src2d = edge_index[0].reshape(ER, EB)
    dst2d = edge_index[1].reshape(ER, EB)
    zeros64 = jnp.zeros((NN, 64), jnp.float32)
    zeros32 = jnp.zeros((NN, 32), jnp.float32)

    degp = _deg_kernel(dst2d).reshape(NC, NN).T
    h1m, xs = pl.pallas_call(
        _tc_a1,
        out_shape=(_f32((NN, 64)), _f32((NN, 64))),
    )(x, W1, Ws1, bs1)
    dinv, hs1 = pl.pallas_call(
        _tc_a2,
        out_shape=(_f32((NN, 1)), _f32((NN, 64))),
    )(h1m, degp)

    acc1 = _scat64(hs1, src2d, dst2d, zeros64)
    hs2, h1 = pl.pallas_call(
        _tc_b,
        out_shape=(_f32((NN, 64)), _f32((NN, 64))),
    )(acc1, hs1, xs, dinv, b1, g1, be1, W2)
    h1s = pl.pallas_call(
        _tc_h1s,
        out_shape=_f32((NN, 32)),
    )(h1, Ws2, bs2)

    acc2 = _scat64(hs2, src2d, dst2d, zeros64)
    hs3 = pl.pallas_call(
        _tc_c,
        out_shape=_f32((NN, 32)),
    )(acc2, hs2, dinv, b2, g2, be2, W3)

    acc3 = _scat32(hs3, src2d, dst2d, zeros32)
    adj, z = pl.pallas_call(
        _tc_decode,
        grid=(_NGI, _NGJ),
        in_specs=[
            pl.BlockSpec((NC * NN, 32), lambda i, j: (0, 0)),
            pl.BlockSpec((NN, 32), lambda i, j: (0, 0)),
            pl.BlockSpec((NN, 1), lambda i, j: (0, 0)),
            pl.BlockSpec((32,), lambda i, j: (0,)),
            pl.BlockSpec((NN, 32), lambda i, j: (0, 0)),
        ],
        out_specs=(
            pl.BlockSpec((_BM, _BN), lambda i, j: (i, j)),
            pl.BlockSpec((NN, 32), lambda i, j: (0, 0)),
        ),
        out_shape=(_f32((NN, NN)), _f32((NN, 32))),
        scratch_shapes=[pltpu.VMEM((NN + _ZPAD, 32), jnp.float32)],
    )(acc3, hs3, dinv, b3, h1s)
    return adj, z


# speedup vs baseline: 1.0058x; 1.0058x over previous
import jax
import jax.numpy as jnp
from jax import lax
from jax.experimental import pallas as pl
from jax.experimental.pallas import tpu as pltpu
from jax.experimental.pallas import tpu_sc as plsc

NN = 10000
EE = 320000
NC = 2
NS = 16
NW = NC * NS
EB = 125
ER = EE // EB
RPT = ER // NW
CH = 632
CHL = NN - 15 * CH

_MESH = plsc.VectorSubcoreMesh(core_axis_name="c", subcore_axis_name="s")


_DCH = 640


def _deg_body(dst_hbm, out_hbm, dst_v, ones_v, chunk_v, deg_sh):
    cid = lax.axis_index("c")
    sid = lax.axis_index("s")
    wid = sid * NC + cid

    for i in range(_DCH // 16):
        chunk_v[pl.ds(i * 16, 16)] = jnp.zeros((16,), jnp.float32)
    pltpu.sync_copy(chunk_v, deg_sh.at[pl.ds(sid * _DCH, _DCH)])
    plsc.subcore_barrier()

    pltpu.sync_copy(dst_hbm.at[pl.ds(wid * RPT, RPT)], dst_v)
    for i in range(8):
        ones_v[pl.ds(i * 16, 16)] = jnp.ones((16,), jnp.float32)

    def body(j, carry):
        pltpu.sync_copy(ones_v.at[pl.ds(0, EB)], deg_sh.at[dst_v.at[j]],
                        add=True)
        return carry

    lax.fori_loop(0, RPT, body, 0)
    plsc.subcore_barrier()

    pltpu.sync_copy(deg_sh.at[pl.ds(sid * _DCH, _DCH)], chunk_v)

    @pl.when(sid < 15)
    def _():
        pltpu.sync_copy(chunk_v, out_hbm.at[pl.ds(cid * NN + sid * _DCH,
                                                  _DCH)])

    @pl.when(sid == 15)
    def _():
        pltpu.sync_copy(chunk_v.at[pl.ds(0, NN - 15 * _DCH)],
                        out_hbm.at[pl.ds(cid * NN + 15 * _DCH,
                                         NN - 15 * _DCH)])


_deg_kernel = pl.kernel(
    _deg_body,
    out_type=jax.ShapeDtypeStruct((NC * NN,), jnp.float32),
    mesh=_MESH,
    scratch_types=[
        pltpu.VMEM((RPT, EB), jnp.int32),
        pltpu.VMEM((128,), jnp.float32),
        pltpu.VMEM((_DCH,), jnp.float32),
        pltpu.VMEM_SHARED((NS * _DCH,), jnp.float32),
    ],
)


_NBT = 8


def _make_scat(width):
    def body(hs_hbm, src_hbm, dst_hbm, zeros_hbm, out_hbm,
             src_v, dst_v, rows_v, gsem, acc_sh):
        cid = lax.axis_index("c")
        sid = lax.axis_index("s")
        wid = sid * NC + cid

        @pl.when(sid < 15)
        def _():
            pltpu.sync_copy(zeros_hbm.at[pl.ds(sid * CH, CH)],
                            acc_sh.at[pl.ds(sid * CH, CH)])

        @pl.when(sid == 15)
        def _():
            pltpu.sync_copy(zeros_hbm.at[pl.ds(15 * CH, CHL)],
                            acc_sh.at[pl.ds(15 * CH, CHL)])

        pltpu.sync_copy(src_hbm.at[pl.ds(wid * RPT, RPT)], src_v)
        pltpu.sync_copy(dst_hbm.at[pl.ds(wid * RPT, RPT)], dst_v)
        plsc.subcore_barrier()

        def gstart(j, b):
            pltpu.async_copy(hs_hbm.at[src_v.at[j]], rows_v.at[b],
                             gsem.at[b])

        def gwait(j, b):
            pltpu.make_async_copy(hs_hbm.at[src_v.at[j]], rows_v.at[b],
                                  gsem.at[b]).wait()

        for b in range(_NBT):
            gstart(b, b)

        def rnd(r, carry):
            base = r * _NBT
            for b in range(_NBT):
                j = base + b
                gwait(j, b)
                pltpu.sync_copy(rows_v.at[b], acc_sh.at[dst_v.at[j]],
                                add=True)
                jn = j + _NBT

                @pl.when(jn < RPT)
                def _():
                    gstart(jn, b)
            return carry

        lax.fori_loop(0, RPT // _NBT, rnd, 0)
        plsc.subcore_barrier()

        @pl.when(sid < 15)
        def _():
            pltpu.sync_copy(acc_sh.at[pl.ds(sid * CH, CH)],
                            out_hbm.at[pl.ds(cid * NN + sid * CH, CH)])

        @pl.when(sid == 15)
        def _():
            pltpu.sync_copy(acc_sh.at[pl.ds(15 * CH, CHL)],
                            out_hbm.at[pl.ds(cid * NN + 15 * CH, CHL)])

    return pl.kernel(
        body,
        out_type=jax.ShapeDtypeStruct((NC * NN, width), jnp.float32),
        mesh=_MESH,
        compiler_params=pltpu.CompilerParams(use_tc_tiling_on_sc=False),
        scratch_types=[
            pltpu.VMEM((RPT, EB), jnp.int32),
            pltpu.VMEM((RPT, EB), jnp.int32),
            pltpu.VMEM((_NBT, EB, width), jnp.float32),
            pltpu.SemaphoreType.DMA((_NBT,)),
            pltpu.VMEM_SHARED((NN, width), jnp.float32),
        ],
    )


_scat64 = _make_scat(64)
_scat32 = _make_scat(32)


def _tc_a1(x_ref, w1_ref, ws1_ref, bs1_ref, h1m_ref, xs_ref):
    h1m_ref[...] = jnp.dot(x_ref[...], w1_ref[...])
    xs_ref[...] = jnp.dot(x_ref[...], ws1_ref[...]) + bs1_ref[...]


def _tc_a2(h1m_ref, degp_ref, dinv_ref, hs1_ref):
    deg = degp_ref[:, 0:1] + degp_ref[:, 1:2] + 1.0
    dinv = lax.rsqrt(deg)
    dinv_ref[...] = dinv
    hs1_ref[...] = h1m_ref[...] * dinv


def _bn(h, g, be):
    m = jnp.mean(h, axis=0, keepdims=True)
    v = jnp.mean((h - m) ** 2, axis=0, keepdims=True)
    return (h - m) * lax.rsqrt(v + 1e-5) * g + be


def _accsum(ref):
    return ref[pl.ds(0, NN), :] + ref[pl.ds(NN, NN), :]


def _tc_b(acc_ref, hs1_ref, xs_ref, dinv_ref, b1_ref, g1_ref, be1_ref,
          w2_ref, ws2_ref, bs2_ref, hs2_ref, h1s_ref):
    dinv = dinv_ref[...]
    conv = dinv * (_accsum(acc_ref) + hs1_ref[...]) + b1_ref[...]
    h = _bn(jnp.maximum(conv, 0.0), g1_ref[...], be1_ref[...])
    h1 = h + xs_ref[...]
    hs2_ref[...] = jnp.dot(h1, w2_ref[...]) * dinv
    h1s_ref[...] = jnp.dot(h1, ws2_ref[...]) + bs2_ref[...]


def _tc_c(acc_ref, hs2_ref, dinv_ref, b2_ref, g2_ref, be2_ref, w3_ref,
          hs3_ref):
    dinv = dinv_ref[...]
    conv = dinv * (_accsum(acc_ref) + hs2_ref[...]) + b2_ref[...]
    h2 = _bn(jnp.maximum(conv, 0.0), g2_ref[...], be2_ref[...])
    hs3_ref[...] = jnp.dot(h2, w3_ref[...]) * dinv


_BM = 1024
_BN = 1024
_NGI = pl.cdiv(NN, _BM)
_NGJ = pl.cdiv(NN, _BN)


_ZPAD = max(_NGI * _BM, _NGJ * _BN) - NN


def _tc_decode(acc_ref, hs3_ref, dinv_ref, b3_ref, h1s_ref,
               adj_ref, zout_ref, z_v):
    i = pl.program_id(0)
    j = pl.program_id(1)

    @pl.when((i == 0) & (j == 0))
    def _():
        z = (dinv_ref[...] * (_accsum(acc_ref) + hs3_ref[...])
             + b3_ref[...] + h1s_ref[...])
        z_v[pl.ds(0, NN), :] = z
        z_v[pl.ds(NN, _ZPAD), :] = jnp.zeros((_ZPAD, 32), jnp.float32)
        zout_ref[...] = z

    zi = z_v[pl.ds(i * _BM, _BM), :]
    zj = z_v[pl.ds(j * _BN, _BN), :]
    prod = lax.dot_general(zi, zj, (((1,), (1,)), ((), ())))
    adj_ref[...] = jax.nn.sigmoid(prod)


def _f32(shape):
    return jax.ShapeDtypeStruct(shape, jnp.float32)


def kernel(x, edge_index, W1, b1, W2, b2, W3, b3, Ws1, bs1, Ws2, bs2,
           g1, be1, g2, be2):
    src2d = edge_index[0].reshape(ER, EB)
    dst2d = edge_index[1].reshape(ER, EB)
    zeros64 = jnp.zeros((NN, 64), jnp.float32)
    zeros32 = jnp.zeros((NN, 32), jnp.float32)

    degp = _deg_kernel(dst2d).reshape(NC, NN).T
    h1m, xs = pl.pallas_call(
        _tc_a1,
        out_shape=(_f32((NN, 64)), _f32((NN, 64))),
    )(x, W1, Ws1, bs1)
    dinv, hs1 = pl.pallas_call(
        _tc_a2,
        out_shape=(_f32((NN, 1)), _f32((NN, 64))),
    )(h1m, degp)

    acc1 = _scat64(hs1, src2d, dst2d, zeros64)
    hs2, h1s = pl.pallas_call(
        _tc_b,
        out_shape=(_f32((NN, 64)), _f32((NN, 32))),
    )(acc1, hs1, xs, dinv, b1, g1, be1, W2, Ws2, bs2)

    acc2 = _scat64(hs2, src2d, dst2d, zeros64)
    hs3 = pl.pallas_call(
        _tc_c,
        out_shape=_f32((NN, 32)),
    )(acc2, hs2, dinv, b2, g2, be2, W3)

    acc3 = _scat32(hs3, src2d, dst2d, zeros32)
    adj, z = pl.pallas_call(
        _tc_decode,
        grid=(_NGI, _NGJ),
        in_specs=[
            pl.BlockSpec((NC * NN, 32), lambda i, j: (0, 0)),
            pl.BlockSpec((NN, 32), lambda i, j: (0, 0)),
            pl.BlockSpec((NN, 1), lambda i, j: (0, 0)),
            pl.BlockSpec((32,), lambda i, j: (0,)),
            pl.BlockSpec((NN, 32), lambda i, j: (0, 0)),
        ],
        out_specs=(
            pl.BlockSpec((_BM, _BN), lambda i, j: (i, j)),
            pl.BlockSpec((NN, 32), lambda i, j: (0, 0)),
        ),
        out_shape=(_f32((NN, NN)), _f32((NN, 32))),
        scratch_shapes=[pltpu.VMEM((NN + _ZPAD, 32), jnp.float32)],
    )(acc3, hs3, dinv, b3, h1s)
    return adj, z


# speedup vs baseline: 1.0077x; 1.0019x over previous
import jax
import jax.numpy as jnp
from jax import lax
from jax.experimental import pallas as pl
from jax.experimental.pallas import tpu as pltpu
from jax.experimental.pallas import tpu_sc as plsc

NN = 10000
EE = 320000
NC = 2
NS = 16
NW = NC * NS
EB = 125
ER = EE // EB
RPT = ER // NW
CH = 632
CHL = NN - 15 * CH

_MESH = plsc.VectorSubcoreMesh(core_axis_name="c", subcore_axis_name="s",
                               num_cores=NC, num_subcores=NS)


_DCH = 640


def _deg_body(dst_hbm, out_hbm, dst_v, ones_v, chunk_v, deg_sh):
    cid = lax.axis_index("c")
    sid = lax.axis_index("s")
    wid = sid * NC + cid

    for i in range(_DCH // 16):
        chunk_v[pl.ds(i * 16, 16)] = jnp.zeros((16,), jnp.float32)
    pltpu.sync_copy(chunk_v, deg_sh.at[pl.ds(sid * _DCH, _DCH)])
    plsc.subcore_barrier()

    pltpu.sync_copy(dst_hbm.at[pl.ds(wid * RPT, RPT)], dst_v)
    for i in range(8):
        ones_v[pl.ds(i * 16, 16)] = jnp.ones((16,), jnp.float32)

    def body(j, carry):
        pltpu.sync_copy(ones_v.at[pl.ds(0, EB)], deg_sh.at[dst_v.at[j]],
                        add=True)
        return carry

    lax.fori_loop(0, RPT, body, 0)
    plsc.subcore_barrier()

    pltpu.sync_copy(deg_sh.at[pl.ds(sid * _DCH, _DCH)], chunk_v)

    @pl.when(sid < 15)
    def _():
        pltpu.sync_copy(chunk_v, out_hbm.at[pl.ds(cid * NN + sid * _DCH,
                                                  _DCH)])

    @pl.when(sid == 15)
    def _():
        pltpu.sync_copy(chunk_v.at[pl.ds(0, NN - 15 * _DCH)],
                        out_hbm.at[pl.ds(cid * NN + 15 * _DCH,
                                         NN - 15 * _DCH)])


_deg_kernel = pl.kernel(
    _deg_body,
    out_type=jax.ShapeDtypeStruct((NC * NN,), jnp.float32),
    mesh=_MESH,
    scratch_types=[
        pltpu.VMEM((RPT, EB), jnp.int32),
        pltpu.VMEM((128,), jnp.float32),
        pltpu.VMEM((_DCH,), jnp.float32),
        pltpu.VMEM_SHARED((NS * _DCH,), jnp.float32),
    ],
)


_NBT = 8


def _make_scat(width):
    def body(hs_hbm, src_hbm, dst_hbm, zeros_hbm, out_hbm,
             src_v, dst_v, rows_v, gsem, acc_sh):
        cid = lax.axis_index("c")
        sid = lax.axis_index("s")
        wid = sid * NC + cid

        @pl.when(sid < 15)
        def _():
            pltpu.sync_copy(zeros_hbm.at[pl.ds(sid * CH, CH)],
                            acc_sh.at[pl.ds(sid * CH, CH)])

        @pl.when(sid == 15)
        def _():
            pltpu.sync_copy(zeros_hbm.at[pl.ds(15 * CH, CHL)],
                            acc_sh.at[pl.ds(15 * CH, CHL)])

        pltpu.sync_copy(src_hbm.at[pl.ds(wid * RPT, RPT)], src_v)
        pltpu.sync_copy(dst_hbm.at[pl.ds(wid * RPT, RPT)], dst_v)
        plsc.subcore_barrier()

        def gstart(j, b):
            pltpu.async_copy(hs_hbm.at[src_v.at[j]], rows_v.at[b],
                             gsem.at[b])

        def gwait(j, b):
            pltpu.make_async_copy(hs_hbm.at[src_v.at[j]], rows_v.at[b],
                                  gsem.at[b]).wait()

        for b in range(_NBT):
            gstart(b, b)

        def rnd(r, carry):
            base = r * _NBT
            for b in range(_NBT):
                j = base + b
                gwait(j, b)
                pltpu.sync_copy(rows_v.at[b], acc_sh.at[dst_v.at[j]],
                                add=True)
                jn = j + _NBT

                @pl.when(jn < RPT)
                def _():
                    gstart(jn, b)
            return carry

        lax.fori_loop(0, RPT // _NBT, rnd, 0)
        plsc.subcore_barrier()

        @pl.when(sid < 15)
        def _():
            pltpu.sync_copy(acc_sh.at[pl.ds(sid * CH, CH)],
                            out_hbm.at[pl.ds(cid * NN + sid * CH, CH)])

        @pl.when(sid == 15)
        def _():
            pltpu.sync_copy(acc_sh.at[pl.ds(15 * CH, CHL)],
                            out_hbm.at[pl.ds(cid * NN + 15 * CH, CHL)])

    return pl.kernel(
        body,
        out_type=jax.ShapeDtypeStruct((NC * NN, width), jnp.float32),
        mesh=_MESH,
        compiler_params=pltpu.CompilerParams(use_tc_tiling_on_sc=False),
        scratch_types=[
            pltpu.VMEM((RPT, EB), jnp.int32),
            pltpu.VMEM((RPT, EB), jnp.int32),
            pltpu.VMEM((_NBT, EB, width), jnp.float32),
            pltpu.SemaphoreType.DMA((_NBT,)),
            pltpu.VMEM_SHARED((NN, width), jnp.float32),
        ],
    )


_scat64 = _make_scat(64)
_scat32 = _make_scat(32)


def _tc_a1(x_ref, w1_ref, ws1_ref, bs1_ref, h1m_ref, xs_ref):
    h1m_ref[...] = jnp.dot(x_ref[...], w1_ref[...])
    xs_ref[...] = jnp.dot(x_ref[...], ws1_ref[...]) + bs1_ref[...]


def _tc_a2(h1m_ref, degp_ref, dinv_ref, hs1_ref):
    deg = degp_ref[:, 0:1] + degp_ref[:, 1:2] + 1.0
    dinv = lax.rsqrt(deg)
    dinv_ref[...] = dinv
    hs1_ref[...] = h1m_ref[...] * dinv


def _bn(h, g, be):
    m = jnp.mean(h, axis=0, keepdims=True)
    v = jnp.mean((h - m) ** 2, axis=0, keepdims=True)
    return (h - m) * lax.rsqrt(v + 1e-5) * g + be


def _accsum(ref):
    return ref[pl.ds(0, NN), :] + ref[pl.ds(NN, NN), :]


def _tc_b(acc_ref, hs1_ref, xs_ref, dinv_ref, b1_ref, g1_ref, be1_ref,
          w2_ref, ws2_ref, bs2_ref, hs2_ref, h1s_ref):
    dinv = dinv_ref[...]
    conv = dinv * (_accsum(acc_ref) + hs1_ref[...]) + b1_ref[...]
    h = _bn(jnp.maximum(conv, 0.0), g1_ref[...], be1_ref[...])
    h1 = h + xs_ref[...]
    hs2_ref[...] = jnp.dot(h1, w2_ref[...]) * dinv
    h1s_ref[...] = jnp.dot(h1, ws2_ref[...]) + bs2_ref[...]


def _tc_c(acc_ref, hs2_ref, dinv_ref, b2_ref, g2_ref, be2_ref, w3_ref,
          hs3_ref):
    dinv = dinv_ref[...]
    conv = dinv * (_accsum(acc_ref) + hs2_ref[...]) + b2_ref[...]
    h2 = _bn(jnp.maximum(conv, 0.0), g2_ref[...], be2_ref[...])
    hs3_ref[...] = jnp.dot(h2, w3_ref[...]) * dinv


_BM = 1024
_BN = 1024
_NGI = pl.cdiv(NN, _BM)
_NGJ = pl.cdiv(NN, _BN)


_ZPAD = max(_NGI * _BM, _NGJ * _BN) - NN


def _tc_decode(acc_ref, hs3_ref, dinv_ref, b3_ref, h1s_ref,
               adj_ref, zout_ref, z_v):
    i = pl.program_id(0)
    j = pl.program_id(1)

    @pl.when((i == 0) & (j == 0))
    def _():
        z = (dinv_ref[...] * (_accsum(acc_ref) + hs3_ref[...])
             + b3_ref[...] + h1s_ref[...])
        z_v[pl.ds(0, NN), :] = z
        z_v[pl.ds(NN, _ZPAD), :] = jnp.zeros((_ZPAD, 32), jnp.float32)
        zout_ref[...] = z

    zi = z_v[pl.ds(i * _BM, _BM), :]
    zj = z_v[pl.ds(j * _BN, _BN), :]
    prod = lax.dot_general(zi, zj, (((1,), (1,)), ((), ())))
    adj_ref[...] = jax.nn.sigmoid(prod)


def _f32(shape):
    return jax.ShapeDtypeStruct(shape, jnp.float32)


def kernel(x, edge_index, W1, b1, W2, b2, W3, b3, Ws1, bs1, Ws2, bs2,
           g1, be1, g2, be2):
    src2d = edge_index[0].reshape(ER, EB)
    dst2d = edge_index[1].reshape(ER, EB)
    zeros64 = jnp.zeros((NN, 64), jnp.float32)
    zeros32 = jnp.zeros((NN, 32), jnp.float32)

    degp = _deg_kernel(dst2d).reshape(NC, NN).T
    h1m, xs = pl.pallas_call(
        _tc_a1,
        out_shape=(_f32((NN, 64)), _f32((NN, 64))),
    )(x, W1, Ws1, bs1)
    dinv, hs1 = pl.pallas_call(
        _tc_a2,
        out_shape=(_f32((NN, 1)), _f32((NN, 64))),
    )(h1m, degp)

    acc1 = _scat64(hs1, src2d, dst2d, zeros64)
    hs2, h1s = pl.pallas_call(
        _tc_b,
        out_shape=(_f32((NN, 64)), _f32((NN, 32))),
    )(acc1, hs1, xs, dinv, b1, g1, be1, W2, Ws2, bs2)

    acc2 = _scat64(hs2, src2d, dst2d, zeros64)
    hs3 = pl.pallas_call(
        _tc_c,
        out_shape=_f32((NN, 32)),
    )(acc2, hs2, dinv, b2, g2, be2, W3)

    acc3 = _scat32(hs3, src2d, dst2d, zeros32)
    adj, z = pl.pallas_call(
        _tc_decode,
        grid=(_NGI, _NGJ),
        in_specs=[
            pl.BlockSpec((NC * NN, 32), lambda i, j: (0, 0)),
            pl.BlockSpec((NN, 32), lambda i, j: (0, 0)),
            pl.BlockSpec((NN, 1), lambda i, j: (0, 0)),
            pl.BlockSpec((32,), lambda i, j: (0,)),
            pl.BlockSpec((NN, 32), lambda i, j: (0, 0)),
        ],
        out_specs=(
            pl.BlockSpec((_BM, _BN), lambda i, j: (i, j)),
            pl.BlockSpec((NN, 32), lambda i, j: (0, 0)),
        ),
        out_shape=(_f32((NN, NN)), _f32((NN, 32))),
        scratch_shapes=[pltpu.VMEM((NN + _ZPAD, 32), jnp.float32)],
    )(acc3, hs3, dinv, b3, h1s)
    return adj, z
